# baseline (speedup 1.0000x reference)
import functools

import jax
import jax.numpy as jnp
from jax import lax
from jax.experimental import pallas as pl
from jax.experimental.pallas import tpu as pltpu
from jax.experimental.pallas import tpu_sc as plsc

B, H, W, K = 4, 512, 512, 8
N = B * K
TILE_H = 256
NT = H // TILE_H
NCH = 16
NBAND = 8
BAND_H = H // NBAND
CH_ROWS = 8
NCHUNK = BAND_H // CH_ROWS


def _tc_body(lbl_ref, xy_ref, bin_ref, xym_ref):
    lbl = lbl_ref[0]
    for k in range(K):
        m = (lbl == (k + 1)).astype(jnp.float32)
        bin_ref[k] = m
        xym_ref[k, 0] = m * xy_ref[0, 0]
        xym_ref[k, 1] = m * xy_ref[0, 1]


def _tc_pass(instance_masks, xy):
    return pl.pallas_call(
        _tc_body,
        grid=(B, NT),
        in_specs=[
            pl.BlockSpec((1, TILE_H, W), lambda b, t: (b, t, 0)),
            pl.BlockSpec((1, 2, TILE_H, W), lambda b, t: (b, 0, t, 0)),
        ],
        out_specs=(
            pl.BlockSpec((K, TILE_H, W), lambda b, t: (b, t, 0)),
            pl.BlockSpec((K, 2, TILE_H, W), lambda b, t: (b, 0, t, 0)),
        ),
        out_shape=(
            jax.ShapeDtypeStruct((N, H, W), jnp.float32),
            jax.ShapeDtypeStruct((N, 2, H, W), jnp.float32),
        ),
    )(instance_masks, xy)


def _sc_sums_body(cat_hbm, lbl_hbm, q_hbm, s_hbm, z_hbm, part_hbm,
                  lbl_buf, cat_buf, ch_buf, acc, out_buf, sem0, sem1):
    info = plsc.get_sparse_core_info()
    nc = info.num_cores
    wid = lax.axis_index("s") * nc + lax.axis_index("c")
    b = wid // NBAND
    t = wid % NBAND
    row0 = t * BAND_H
    sems = (sem0, sem1)
    lane = lax.broadcasted_iota(jnp.int32, (16,), 0)
    ones = jnp.ones((16,), jnp.float32)

    for c in range(10):
        for k in range(K):
            acc[c, k] = jnp.zeros((16,), jnp.float32)

    def issue(g):
        p = g % 2
        r0 = row0 + g * CH_ROWS
        sl = pl.ds(r0, CH_ROWS)
        sem = sems[p]
        d = [pltpu.async_copy(lbl_hbm.at[b, sl], lbl_buf.at[p], sem),
             pltpu.async_copy(cat_hbm.at[b, sl], cat_buf.at[p], sem)]
        for c in range(4):
            d.append(pltpu.async_copy(q_hbm.at[b, c, sl], ch_buf.at[p, c], sem))
        for c in range(3):
            d.append(pltpu.async_copy(s_hbm.at[b, c, sl], ch_buf.at[p, 4 + c], sem))
        d.append(pltpu.async_copy(z_hbm.at[b, sl], ch_buf.at[p, 7], sem))
        return d

    pending = issue(0)
    for g in range(NCHUNK):
        nxt = issue(g + 1) if g + 1 < NCHUNK else []
        for h in pending:
            h.wait()
        pending = nxt
        p = g % 2
        for r in range(CH_ROWS):
            def col_body(j, _, p=p, r=r):
                col = j * 16
                cs = pl.ds(col, 16)
                km1 = lbl_buf[p, r, cs] - 1
                for c in range(8):
                    plsc.addupdate_scatter(
                        acc, [jnp.full((16,), c, jnp.int32), km1, lane],
                        ch_buf[p, c, r, cs])
                plsc.addupdate_scatter(
                    acc, [jnp.full((16,), 8, jnp.int32), km1, lane], ones)
                plsc.addupdate_scatter(
                    acc, [jnp.full((16,), 9, jnp.int32), km1, lane],
                    cat_buf[p, r, cs].astype(jnp.float32))
                return _
            lax.fori_loop(0, W // 16, col_body, None)

    for k in range(K):
        row = jnp.zeros((16,), jnp.float32)
        for c in range(10):
            s = jnp.sum(acc[c, k])
            row = jnp.where(lane == c, s, row)
        out_buf[k] = row
    pltpu.sync_copy(out_buf, part_hbm.at[b, t])


@functools.cache
def _sc_sums():
    return pl.kernel(
        _sc_sums_body,
        out_type=jax.ShapeDtypeStruct((B, NBAND, K, NCH), jnp.float32),
        mesh=plsc.VectorSubcoreMesh(core_axis_name="c", subcore_axis_name="s"),
        scratch_types=[
            pltpu.VMEM((2, CH_ROWS, W), jnp.int32),
            pltpu.VMEM((2, CH_ROWS, W), jnp.int32),
            pltpu.VMEM((2, 8, CH_ROWS, W), jnp.float32),
            pltpu.VMEM((10, K, 16), jnp.float32),
            pltpu.VMEM((K, NCH), jnp.float32),
            pltpu.SemaphoreType.DMA,
            pltpu.SemaphoreType.DMA,
        ],
        compiler_params=pltpu.CompilerParams(needs_layout_passes=False),
    )


def _sc_finish_body(part_hbm, out_hbm, part_v, out_v):
    info = plsc.get_sparse_core_info()
    nc = info.num_cores
    wid = lax.axis_index("s") * nc + lax.axis_index("c")
    b = wid // K
    k = wid % K
    pltpu.sync_copy(part_hbm.at[b], part_v)
    acc = part_v[0, k]
    for t in range(1, NBAND):
        acc = acc + part_v[t, k]
    lane = lax.broadcasted_iota(jnp.int32, (16,), 0)
    size = jnp.sum(jnp.where(lane == 8, acc, 0.0))
    mean = acc / size
    nrm2 = jnp.sum(jnp.where(lane < 4, mean * mean, 0.0))
    nv = jnp.where(lane < 16, nrm2, 0.0)
    bits = lax.bitcast_convert_type(nv, jnp.int32)
    y = lax.bitcast_convert_type(
        jnp.int32(0x5F3759DF) - lax.shift_right_arithmetic(bits, 1),
        jnp.float32)
    for _ in range(3):
        y = y * (1.5 - 0.5 * nv * y * y)
    cls = jnp.sum(jnp.where(lane == 9, mean, 0.0))
    res = jnp.where(lane < 4, mean * y, mean)
    res = jnp.where(lane == 7, jnp.exp(mean), res)
    res = jnp.where(lane == 8, cls, res)
    res = jnp.where(lane >= 9, 0.0, res)
    out_v[...] = res
    pltpu.sync_copy(out_v, out_hbm.at[wid])


@functools.cache
def _sc_finish():
    return pl.kernel(
        _sc_finish_body,
        out_type=jax.ShapeDtypeStruct((N, 16), jnp.float32),
        mesh=plsc.VectorSubcoreMesh(core_axis_name="c", subcore_axis_name="s"),
        scratch_types=[
            pltpu.VMEM((NBAND, K, NCH), jnp.float32),
            pltpu.VMEM((16,), jnp.float32),
        ],
        compiler_params=pltpu.CompilerParams(needs_layout_passes=False),
    )


def kernel(cat_mask, instance_masks, quaternion, scales, xy, z):
    cat_mask = cat_mask.astype(jnp.int32)
    instance_masks = instance_masks.astype(jnp.int32)
    binary_flat, xy_masked = _tc_pass(instance_masks, xy)
    partials = _sc_sums()(cat_mask, instance_masks, quaternion, scales, z)
    fin = _sc_finish()(partials)
    quat_agg = fin[:, 0:4]
    scales_agg = fin[:, 4:7]
    z_agg = fin[:, 7:8]
    class_ids = fin[:, 8].astype(cat_mask.dtype)
    sample_ids = jnp.repeat(jnp.arange(B, dtype=class_ids.dtype), K)
    return (class_ids, sample_ids, binary_flat, quat_agg, scales_agg,
            xy_masked, z_agg)


# speedup vs baseline: 2.0042x; 2.0042x over previous
import functools

import jax
import jax.numpy as jnp
from jax import lax
from jax.experimental import pallas as pl
from jax.experimental.pallas import tpu as pltpu
from jax.experimental.pallas import tpu_sc as plsc

B, H, W, K = 4, 512, 512, 8
N = B * K
TILE_H = 256
NT = H // TILE_H
NCH = 16


def _tc_body(cat_ref, lbl_ref, q_ref, s_ref, xy_ref, z_ref,
             bin_ref, xym_ref, part_ref):
    lbl = lbl_ref[0]
    cat = cat_ref[0].astype(jnp.float32)
    ones = jnp.ones_like(cat)
    feats = jnp.concatenate(
        [q_ref[0], s_ref[0], z_ref[0][None], ones[None], cat[None]], axis=0)
    rows = []
    for k in range(K):
        m = (lbl == (k + 1)).astype(jnp.float32)
        bin_ref[k] = m
        xym_ref[k, 0] = m * xy_ref[0, 0]
        xym_ref[k, 1] = m * xy_ref[0, 1]
        rows.append(jnp.sum(m[None] * feats, axis=(1, 2)))
    part = jnp.stack(rows)
    part_ref[0, 0] = jnp.concatenate(
        [part, jnp.zeros((K, NCH - part.shape[1]), jnp.float32)], axis=1)


def _tc_pass(cat_mask, instance_masks, quaternion, scales, xy, z):
    return pl.pallas_call(
        _tc_body,
        grid=(B, NT),
        in_specs=[
            pl.BlockSpec((1, TILE_H, W), lambda b, t: (b, t, 0)),
            pl.BlockSpec((1, TILE_H, W), lambda b, t: (b, t, 0)),
            pl.BlockSpec((1, 4, TILE_H, W), lambda b, t: (b, 0, t, 0)),
            pl.BlockSpec((1, 3, TILE_H, W), lambda b, t: (b, 0, t, 0)),
            pl.BlockSpec((1, 2, TILE_H, W), lambda b, t: (b, 0, t, 0)),
            pl.BlockSpec((1, TILE_H, W), lambda b, t: (b, t, 0)),
        ],
        out_specs=(
            pl.BlockSpec((K, TILE_H, W), lambda b, t: (b, t, 0)),
            pl.BlockSpec((K, 2, TILE_H, W), lambda b, t: (b, 0, t, 0)),
            pl.BlockSpec((1, 1, K, NCH), lambda b, t: (b, t, 0, 0)),
        ),
        out_shape=(
            jax.ShapeDtypeStruct((N, H, W), jnp.float32),
            jax.ShapeDtypeStruct((N, 2, H, W), jnp.float32),
            jax.ShapeDtypeStruct((B, NT, K, NCH), jnp.float32),
        ),
    )(cat_mask, instance_masks, quaternion, scales, xy, z)


def _sc_finish_body(part_hbm, out_hbm, part_v, out_v):
    info = plsc.get_sparse_core_info()
    nc = info.num_cores
    wid = lax.axis_index("s") * nc + lax.axis_index("c")
    b = wid // K
    k = wid % K
    pltpu.sync_copy(part_hbm.at[b], part_v)
    acc = part_v[0, k]
    for t in range(1, NT):
        acc = acc + part_v[t, k]
    lane = lax.broadcasted_iota(jnp.int32, (16,), 0)
    size = jnp.sum(jnp.where(lane == 8, acc, 0.0))
    mean = acc / size
    nrm2 = jnp.sum(jnp.where(lane < 4, mean * mean, 0.0))
    nv = jnp.where(lane < 16, nrm2, 0.0)
    bits = lax.bitcast_convert_type(nv, jnp.int32)
    y = lax.bitcast_convert_type(
        jnp.int32(0x5F3759DF) - lax.shift_right_arithmetic(bits, 1),
        jnp.float32)
    for _ in range(3):
        y = y * (1.5 - 0.5 * nv * y * y)
    cls = jnp.sum(jnp.where(lane == 9, mean, 0.0))
    res = jnp.where(lane < 4, mean * y, mean)
    res = jnp.where(lane == 7, jnp.exp(mean), res)
    res = jnp.where(lane == 8, cls, res)
    res = jnp.where(lane >= 9, 0.0, res)
    out_v[...] = res
    pltpu.sync_copy(out_v, out_hbm.at[wid])


@functools.cache
def _sc_finish():
    return pl.kernel(
        _sc_finish_body,
        out_type=jax.ShapeDtypeStruct((N, 16), jnp.float32),
        mesh=plsc.VectorSubcoreMesh(core_axis_name="c", subcore_axis_name="s"),
        scratch_types=[
            pltpu.VMEM((NT, K, NCH), jnp.float32),
            pltpu.VMEM((16,), jnp.float32),
        ],
        compiler_params=pltpu.CompilerParams(needs_layout_passes=False),
    )


def kernel(cat_mask, instance_masks, quaternion, scales, xy, z):
    cat_mask = cat_mask.astype(jnp.int32)
    instance_masks = instance_masks.astype(jnp.int32)
    binary_flat, xy_masked, partials = _tc_pass(
        cat_mask, instance_masks, quaternion, scales, xy, z)
    fin = _sc_finish()(partials)
    quat_agg = fin[:, 0:4]
    scales_agg = fin[:, 4:7]
    z_agg = fin[:, 7:8]
    class_ids = fin[:, 8].astype(cat_mask.dtype)
    sample_ids = jnp.repeat(jnp.arange(B, dtype=class_ids.dtype), K)
    return (class_ids, sample_ids, binary_flat, quat_agg, scales_agg,
            xy_masked, z_agg)
